# Initial kernel scaffold; baseline (speedup 1.0000x reference)
import jax
import jax.numpy as jnp
from jax import lax
from jax.experimental import pallas as pl
from jax.experimental.pallas import tpu as pltpu
from jax.experimental.pallas import tpu_sc as plsc

N = 10000
E = 320000
B = 64
D = 128
NC = 10

NP_ = 10016
DC = 144
NWK = 32
CHK = 81
K = 128
EW = CHK * K
EP = NWK * EW
RPS = NP_ // 16


def _tc_fwd_body(hin_ref, w_ref, asrc_ref, adst_ref, h_ref, sd_ref):
    h = jnp.dot(hin_ref[...], w_ref[...], preferred_element_type=jnp.float32)
    h_ref[:, 0:D] = h
    pad_cols = lax.broadcasted_iota(jnp.int32, (NP_, DC - D), 1)
    h_ref[:, D:DC] = jnp.where(pad_cols == 0, 1.0, 0.0)
    sd_ref[0, :] = jnp.sum(h * asrc_ref[...], axis=1)
    sd_ref[1, :] = jnp.sum(h * adst_ref[...], axis=1)


def _tc_fwd(hin, w, asrc, adst):
    return pl.pallas_call(
        _tc_fwd_body,
        out_shape=(
            jax.ShapeDtypeStruct((NP_, DC), jnp.float32),
            jax.ShapeDtypeStruct((2, NP_), jnp.float32),
        ),
    )(hin, w, asrc.reshape(1, D), adst.reshape(1, D))


def _sc_edge_body(h_hbm, sd_hbm, src_hbm, dst_hbm, zr_hbm, out_hbm,
                  s_v, d_v, src_v, dst_v, w_v, rows_v, out_sh):
    cid = lax.axis_index("c")
    sid = lax.axis_index("s")
    wid = sid * 2 + cid

    pltpu.sync_copy(sd_hbm.at[0], s_v)
    pltpu.sync_copy(sd_hbm.at[1], d_v)
    pltpu.sync_copy(src_hbm.at[wid], src_v)
    pltpu.sync_copy(dst_hbm.at[wid], dst_v)
    pltpu.sync_copy(zr_hbm.at[pl.ds(sid * RPS, RPS)],
                    out_sh.at[pl.ds(sid * RPS, RPS)])
    plsc.subcore_barrier()

    @pl.loop(0, CHK)
    def _chunk(k):
        @pl.loop(0, K, step=16)
        def _att(j):
            si = src_v[k, pl.ds(j, 16)]
            di = dst_v[k, pl.ds(j, 16)]
            sv = plsc.load_gather(s_v, [si])
            dv = plsc.load_gather(d_v, [di])
            e = sv + dv
            e = jnp.maximum(e, 0.2 * e)
            w_v[pl.ds(j, 16)] = jnp.exp(e)

        pltpu.sync_copy(h_hbm.at[src_v.at[k]], rows_v)

        @pl.loop(0, K)
        def _scale(i):
            a = w_v[i]
            for j in range(DC // 16):
                sl = pl.ds(j * 16, 16)
                rows_v[i, sl] = rows_v[i, sl] * a

        pltpu.sync_copy(rows_v, out_sh.at[dst_v.at[k]], add=True)

    plsc.subcore_barrier()
    pltpu.sync_copy(out_sh.at[pl.ds(sid * RPS, RPS)],
                    out_hbm.at[cid].at[pl.ds(sid * RPS, RPS)])


def _sc_edge(h144, sd, src3, dst3, zr):
    mesh = plsc.VectorSubcoreMesh(core_axis_name="c", subcore_axis_name="s")
    kfn = pl.kernel(
        _sc_edge_body,
        out_type=jax.ShapeDtypeStruct((2, NP_, DC), jnp.float32),
        mesh=mesh,
        scratch_types=[
            pltpu.VMEM((NP_,), jnp.float32),
            pltpu.VMEM((NP_,), jnp.float32),
            pltpu.VMEM((CHK, K), jnp.int32),
            pltpu.VMEM((CHK, K), jnp.int32),
            pltpu.VMEM((K,), jnp.float32),
            pltpu.VMEM((K, DC), jnp.float32),
            pltpu.VMEM_SHARED((NP_, DC), jnp.float32),
        ],
    )
    return kfn(h144, sd, src3, dst3, zr)


def _tc_post_body(ep_ref, b_ref, batch_ref, h_ref, pool_ref):
    num = ep_ref[0, :, 0:D] + ep_ref[1, :, 0:D]
    den = ep_ref[0, :, D:D + 1] + ep_ref[1, :, D:D + 1]
    h = jnp.maximum(num / (den + 1e-16) + b_ref[...], 0.0)
    h_ref[...] = h
    batch = batch_ref[...]

    def seg_body(seg, carry):
        m = batch == seg
        mx = jnp.max(jnp.where(m, h, -jnp.inf), axis=0, keepdims=True)
        sm = jnp.sum(jnp.where(m, h, 0.0), axis=0, keepdims=True)
        cnt = jnp.sum(jnp.where(m, 1.0, 0.0))
        mx = jnp.where(cnt > 0, mx, 0.0)
        mean = sm / jnp.maximum(cnt, 1.0)
        pool_ref[pl.ds(seg, 1), 0:D] = mx
        pool_ref[pl.ds(seg, 1), D:2 * D] = mean
        return carry

    lax.fori_loop(0, B, seg_body, 0)


def _tc_post(ep, b, batch2d):
    return pl.pallas_call(
        _tc_post_body,
        out_shape=(
            jax.ShapeDtypeStruct((NP_, D), jnp.float32),
            jax.ShapeDtypeStruct((B, 2 * D), jnp.float32),
        ),
    )(ep, b.reshape(1, D), batch2d)


def _tc_head_body(p1_ref, p2_ref, p3_ref, l1w_ref, l1b_ref, l2w_ref, l2b_ref,
                  l3w_ref, l3b_ref, out_ref):
    g = p1_ref[...] + p2_ref[...] + p3_ref[...]
    g = jnp.maximum(jnp.dot(g, l1w_ref[...], preferred_element_type=jnp.float32)
                    + l1b_ref[...], 0.0)
    g = jnp.maximum(jnp.dot(g, l2w_ref[...], preferred_element_type=jnp.float32)
                    + l2b_ref[...], 0.0)
    lg = jnp.dot(g, l3w_ref[...], preferred_element_type=jnp.float32) + l3b_ref[...]
    cols = lax.broadcasted_iota(jnp.int32, (B, 128), 1)
    valid = cols < NC
    mx = jnp.max(jnp.where(valid, lg, -jnp.inf), axis=1, keepdims=True)
    ex = jnp.where(valid, jnp.exp(lg - mx), 0.0)
    lse = jnp.log(jnp.sum(ex, axis=1, keepdims=True))
    out_ref[...] = lg - mx - lse


def _tc_head(p1, p2, p3, l1w, l1b, l2w, l2b, l3wp, l3bp):
    return pl.pallas_call(
        _tc_head_body,
        out_shape=jax.ShapeDtypeStruct((B, 128), jnp.float32),
    )(p1, p2, p3, l1w, l1b.reshape(1, D), l2w, l2b.reshape(1, D // 2),
      l3wp, l3bp)


def kernel(x, pos, edge_index, batch, W1, asrc1, adst1, b1, W2, asrc2, adst2,
           b2, W3, asrc3, adst3, b3, L1w, L1b, L2w, L2b, L3w, L3b):
    loops = jnp.arange(N, dtype=jnp.int32)
    pad_e = jnp.full((EP - E - N,), N, jnp.int32)
    src3 = jnp.concatenate([edge_index[0], loops, pad_e]).reshape(NWK, CHK, K)
    dst3 = jnp.concatenate([edge_index[1], loops, pad_e]).reshape(NWK, CHK, K)
    batch2d = jnp.concatenate(
        [batch, jnp.full((NP_ - N,), B, jnp.int32)]).reshape(NP_, 1)
    hin = jnp.pad(jnp.concatenate([x, pos], axis=1), ((0, NP_ - N), (0, 0)))
    zr = jnp.zeros((NP_, DC), jnp.float32)
    l3wp = jnp.pad(L3w, ((0, 0), (0, 128 - NC)))
    l3bp = jnp.pad(L3b, (0, 128 - NC)).reshape(1, 128)

    h, sd = _tc_fwd(hin, W1, asrc1, adst1)
    ep = _sc_edge(h, sd, src3, dst3, zr)
    hr, p1 = _tc_post(ep, b1, batch2d)

    h, sd = _tc_fwd(hr, W2, asrc2, adst2)
    ep = _sc_edge(h, sd, src3, dst3, zr)
    hr, p2 = _tc_post(ep, b2, batch2d)

    h, sd = _tc_fwd(hr, W3, asrc3, adst3)
    ep = _sc_edge(h, sd, src3, dst3, zr)
    hr, p3 = _tc_post(ep, b3, batch2d)

    out = _tc_head(p1, p2, p3, L1w, L1b, L2w, L2b, l3wp, l3bp)
    return out[:, :NC]


# speedup vs baseline: 19.9088x; 19.9088x over previous
import dataclasses

import jax
import jax.numpy as jnp
from jax import lax
from jax.experimental import pallas as pl
from jax.experimental.pallas import tpu as pltpu
from jax.experimental.pallas import tpu_sc as plsc

N = 10000
E = 320000
B = 64
D = 128
NC = 10

NP_ = 10112
HD = 64
DC = 80
NSUB = 16
CHK = 162
K = 128
EP = NSUB * CHK * K
RPS = NP_ // 16


def _tc_fwd_body(hin_ref, w_ref, asrc_ref, adst_ref, h_ref, sd_ref):
    h = jnp.dot(hin_ref[...], w_ref[...], preferred_element_type=jnp.float32)
    h_ref[0, :, 0:HD] = h[:, 0:HD]
    h_ref[1, :, 0:HD] = h[:, HD:D]
    pad_cols = lax.broadcasted_iota(jnp.int32, (NP_, DC - HD), 1)
    h_ref[0, :, HD:DC] = jnp.where(pad_cols == 0, 1.0, 0.0)
    h_ref[1, :, HD:DC] = jnp.zeros((NP_, DC - HD), jnp.float32)
    sd_ref[0, :] = jnp.sum(h * asrc_ref[...], axis=1)
    sd_ref[1, :] = jnp.sum(h * adst_ref[...], axis=1)


def _tc_fwd(hin, w, asrc, adst):
    return pl.pallas_call(
        _tc_fwd_body,
        out_shape=(
            jax.ShapeDtypeStruct((2, NP_, DC), jnp.float32),
            jax.ShapeDtypeStruct((2, NP_), jnp.float32),
        ),
    )(hin, w, asrc.reshape(1, D), adst.reshape(1, D))


def _sc_edge_body(h_hbm, sd_hbm, src_hbm, dst_hbm, zr_hbm, out_hbm,
                  s_v, d_v, src_v, dst_v, w_v, rows_v, out_sh):
    cid = lax.axis_index("c")
    sid = lax.axis_index("s")

    pltpu.sync_copy(sd_hbm.at[0], s_v)
    pltpu.sync_copy(sd_hbm.at[1], d_v)
    pltpu.sync_copy(src_hbm.at[sid], src_v)
    pltpu.sync_copy(dst_hbm.at[sid], dst_v)
    pltpu.sync_copy(zr_hbm.at[pl.ds(sid * RPS, RPS)],
                    out_sh.at[pl.ds(sid * RPS, RPS)])
    plsc.subcore_barrier()

    @pl.loop(0, CHK)
    def _chunk(k):
        @pl.loop(0, K, step=16)
        def _att(j):
            si = src_v[k, pl.ds(j, 16)]
            di = dst_v[k, pl.ds(j, 16)]
            sv = plsc.load_gather(s_v, [si])
            dv = plsc.load_gather(d_v, [di])
            e = sv + dv
            e = jnp.maximum(e, 0.2 * e)
            w_v[pl.ds(j, 16)] = jnp.exp(e)

        pltpu.sync_copy(h_hbm.at[cid].at[src_v.at[k]], rows_v)

        @pl.loop(0, K, step=16)
        def _scale(g):
            w16 = w_v[pl.ds(g, 16)]
            for t in range(16):
                a = w16[t]
                for j in range(DC // 16):
                    sl = pl.ds(j * 16, 16)
                    rows_v[g + t, sl] = rows_v[g + t, sl] * a

        pltpu.sync_copy(rows_v, out_sh.at[dst_v.at[k]], add=True)

    plsc.subcore_barrier()
    pltpu.sync_copy(out_sh.at[pl.ds(sid * RPS, RPS)],
                    out_hbm.at[cid].at[pl.ds(sid * RPS, RPS)])


def _sc_compiler_params():
    cp = pltpu.CompilerParams()
    fields = pltpu.CompilerParams.__dataclass_fields__
    if "needs_layout_passes" in fields:
        cp = dataclasses.replace(cp, needs_layout_passes=False)
    if "use_tc_tiling_on_sc" in fields:
        cp = dataclasses.replace(cp, use_tc_tiling_on_sc=False)
    return cp


def _sc_edge(h_parts, sd, src3, dst3, zr):
    mesh = plsc.VectorSubcoreMesh(core_axis_name="c", subcore_axis_name="s")
    kfn = pl.kernel(
        _sc_edge_body,
        out_type=jax.ShapeDtypeStruct((2, NP_, DC), jnp.float32),
        mesh=mesh,
        scratch_types=[
            pltpu.VMEM((NP_,), jnp.float32),
            pltpu.VMEM((NP_,), jnp.float32),
            pltpu.VMEM((CHK, K), jnp.int32),
            pltpu.VMEM((CHK, K), jnp.int32),
            pltpu.VMEM((K,), jnp.float32),
            pltpu.VMEM((K, DC), jnp.float32),
            pltpu.VMEM_SHARED((NP_, DC), jnp.float32),
        ],
        compiler_params=_sc_compiler_params(),
    )
    return kfn(h_parts, sd, src3, dst3, zr)


def _tc_norm_body(ep_ref, b_ref, batchrow_ref, h_ref, pmean_ref):
    num = jnp.concatenate([ep_ref[0, :, 0:HD], ep_ref[1, :, 0:HD]], axis=1)
    den = ep_ref[0, :, HD:HD + 1]
    h = jnp.maximum(num / (den + 1e-16) + b_ref[...], 0.0)
    h_ref[...] = h
    segs = lax.broadcasted_iota(jnp.int32, (B, NP_), 0)
    mt = jnp.where(segs == batchrow_ref[...], 1.0, 0.0)
    s = jnp.dot(mt, h, preferred_element_type=jnp.float32)
    cnt = jnp.dot(mt, jnp.ones((NP_, 8), jnp.float32),
                  preferred_element_type=jnp.float32)[:, 0:1]
    pmean_ref[...] = s / jnp.maximum(cnt, 1.0)


def _tc_maxpool_body(h_ref, batch_ref, pmax_ref):
    seg = pl.program_id(0)
    m = batch_ref[...] == seg
    pmax_ref[0, 0, :] = jnp.max(jnp.where(m, h_ref[...], 0.0), axis=0)


def _tc_post(ep, b, batch2d, batchrow):
    hr, pmean = pl.pallas_call(
        _tc_norm_body,
        out_shape=(
            jax.ShapeDtypeStruct((NP_, D), jnp.float32),
            jax.ShapeDtypeStruct((B, D), jnp.float32),
        ),
    )(ep, b.reshape(1, D), batchrow)
    pmax = pl.pallas_call(
        _tc_maxpool_body,
        grid=(B,),
        in_specs=[
            pl.BlockSpec((NP_, D), lambda s: (0, 0)),
            pl.BlockSpec((NP_, 1), lambda s: (0, 0)),
        ],
        out_specs=pl.BlockSpec((1, 1, D), lambda s: (s, 0, 0)),
        out_shape=jax.ShapeDtypeStruct((B, 1, D), jnp.float32),
    )(hr, batch2d)
    pool = jnp.concatenate([pmax.reshape(B, D), pmean], axis=1)
    return hr, pool


def _tc_head_body(p1_ref, p2_ref, p3_ref, l1w_ref, l1b_ref, l2w_ref, l2b_ref,
                  l3w_ref, l3b_ref, out_ref):
    g = p1_ref[...] + p2_ref[...] + p3_ref[...]
    g = jnp.maximum(jnp.dot(g, l1w_ref[...], preferred_element_type=jnp.float32)
                    + l1b_ref[...], 0.0)
    g = jnp.maximum(jnp.dot(g, l2w_ref[...], preferred_element_type=jnp.float32)
                    + l2b_ref[...], 0.0)
    lg = jnp.dot(g, l3w_ref[...], preferred_element_type=jnp.float32) + l3b_ref[...]
    cols = lax.broadcasted_iota(jnp.int32, (B, 128), 1)
    valid = cols < NC
    mx = jnp.max(jnp.where(valid, lg, -jnp.inf), axis=1, keepdims=True)
    ex = jnp.where(valid, jnp.exp(lg - mx), 0.0)
    lse = jnp.log(jnp.sum(ex, axis=1, keepdims=True))
    out_ref[...] = lg - mx - lse


def _tc_head(p1, p2, p3, l1w, l1b, l2w, l2b, l3wp, l3bp):
    return pl.pallas_call(
        _tc_head_body,
        out_shape=jax.ShapeDtypeStruct((B, 128), jnp.float32),
    )(p1, p2, p3, l1w, l1b.reshape(1, D), l2w, l2b.reshape(1, D // 2),
      l3wp, l3bp)


def kernel(x, pos, edge_index, batch, W1, asrc1, adst1, b1, W2, asrc2, adst2,
           b2, W3, asrc3, adst3, b3, L1w, L1b, L2w, L2b, L3w, L3b):
    loops = jnp.arange(N, dtype=jnp.int32)
    pad_e = jnp.full((EP - E - N,), N, jnp.int32)
    src3 = jnp.concatenate([edge_index[0], loops, pad_e]).reshape(NSUB, CHK, K)
    dst3 = jnp.concatenate([edge_index[1], loops, pad_e]).reshape(NSUB, CHK, K)
    batch_p = jnp.concatenate([batch, jnp.full((NP_ - N,), B, jnp.int32)])
    batch2d = batch_p.reshape(NP_, 1)
    batchrow = batch_p.reshape(1, NP_)
    hin = jnp.pad(jnp.concatenate([x, pos], axis=1), ((0, NP_ - N), (0, 0)))
    zr = jnp.zeros((NP_, DC), jnp.float32)
    l3wp = jnp.pad(L3w, ((0, 0), (0, 128 - NC)))
    l3bp = jnp.pad(L3b, (0, 128 - NC)).reshape(1, 128)

    h, sd = _tc_fwd(hin, W1, asrc1, adst1)
    ep = _sc_edge(h, sd, src3, dst3, zr)
    hr, p1 = _tc_post(ep, b1, batch2d, batchrow)

    h, sd = _tc_fwd(hr, W2, asrc2, adst2)
    ep = _sc_edge(h, sd, src3, dst3, zr)
    hr, p2 = _tc_post(ep, b2, batch2d, batchrow)

    h, sd = _tc_fwd(hr, W3, asrc3, adst3)
    ep = _sc_edge(h, sd, src3, dst3, zr)
    hr, p3 = _tc_post(ep, b3, batch2d, batchrow)

    out = _tc_head(p1, p2, p3, L1w, L1b, L2w, L2b, l3wp, l3bp)
    return out[:, :NC]
